# baseline (speedup 1.0000x reference)
import functools

import jax
import jax.numpy as jnp
from jax import lax
from jax.experimental import pallas as pl
from jax.experimental.pallas import tpu as pltpu
from jax.experimental.pallas import tpu_sc as plsc

_NLOC = 10000
_NNEI = 32
_NDIM = 128
_EDIM = 16
_AXIS = 4

_NW = 32
_ROWS_PER_W = (_NLOC * _NNEI) // _NW
_CHUNK = 80
_NG = _ROWS_PER_W // _CHUNK


def _sc_gather(table, idx3):
    mesh = plsc.VectorSubcoreMesh(core_axis_name="c", subcore_axis_name="s")

    def body(table_hbm, idx_hbm, out_hbm, idx_v, rows_v, gsem, wsem):
        wid = lax.axis_index("s") * 2 + lax.axis_index("c")
        base = wid * _NG
        pltpu.sync_copy(idx_hbm.at[wid], idx_v)
        pltpu.async_copy(table_hbm.at[idx_v.at[0]], rows_v.at[0], gsem)

        def step(g, carry):
            p = lax.rem(g, 2)
            q = lax.rem(g + 1, 2)

            @pl.when(g >= 1)
            def _():
                pltpu.make_async_copy(rows_v.at[q], out_hbm.at[base + g - 1],
                                      wsem).wait()

            @pl.when(g + 1 < _NG)
            def _():
                pltpu.async_copy(table_hbm.at[idx_v.at[g + 1]], rows_v.at[q],
                                 gsem)

            pltpu.make_async_copy(table_hbm.at[idx_v.at[g]], rows_v.at[p],
                                  gsem).wait()
            pltpu.async_copy(rows_v.at[p], out_hbm.at[base + g], wsem)
            return carry

        lax.fori_loop(0, _NG, step, 0, unroll=False)
        last = _NG - 1
        pltpu.make_async_copy(rows_v.at[last % 2], out_hbm.at[base + last],
                              wsem).wait()

    f = pl.kernel(
        body,
        out_type=jax.ShapeDtypeStruct((_NW * _NG, _CHUNK, _NDIM),
                                      jnp.float32),
        mesh=mesh,
        scratch_types=[
            pltpu.VMEM((_NG, _CHUNK), jnp.int32),
            pltpu.VMEM((2, _CHUNK, _NDIM), jnp.float32),
            pltpu.SemaphoreType.DMA,
            pltpu.SemaphoreType.DMA,
        ],
    )
    return f(table, idx3)


def _silu(x):
    return x / (1.0 + jnp.exp(-x))


_BLK = 200


def _mm(a, b):
    return jax.lax.dot_general(a, b, (((a.ndim - 1,), (0,)), ((), ())),
                               preferred_element_type=jnp.float32)


def _hg(ebd, h2c, swc, blk, d):
    out = []
    for t in range(3):
        w_t = h2c[:, t:t + 1] * swc
        out.append(jnp.sum((ebd * w_t).reshape(blk, _NNEI, d), axis=1)
                   * (1.0 / _NNEI))
    return out


def _grrg_matmul(hg, d, w_ref, sym_pre):
    iota = lax.broadcasted_iota(jnp.int32, (d, d), 0)
    for a in range(_AXIS):
        ea = (iota == a).astype(jnp.float32)
        g_a = sum(hg[t] * _mm(hg[t], ea) for t in range(3)) * (1.0 / 3.0)
        sym_pre = sym_pre + _mm(g_a, w_ref[a])
    return sym_pre


def _tc1_body(node_ref, e_ref, h2_ref, sw_ref,
              w_self_ref, b_self_ref, w_se_ref, b_sym_ref,
              w_ne_ref, w_es_ref, res_n_ref,
              nacc_ref, sympre_ref, ane_ref, aes_ref):
    node = node_ref[...]
    e3 = e_ref[...]
    h2c = h2_ref[...]
    swc = sw_ref[...]
    blk = node.shape[0]
    e = e3.reshape(blk * _NNEI, _EDIM)

    node_self = _silu(_mm(node, w_self_ref[...]) + b_self_ref[...])
    nacc_ref[...] = node + res_n_ref[0] * node_self

    hg_e = _hg(e, h2c, swc, blk, _EDIM)
    sympre_ref[...] = _grrg_matmul(hg_e, _EDIM, w_se_ref, b_sym_ref[...])

    ane_ref[...] = _mm(node, w_ne_ref[0:_NDIM])
    aes_ref[...] = _mm(node, w_es_ref[0:_NDIM])


def _tc2_body(g_ref, e_ref, h2_ref, sw_ref,
              nacc_ref, sympre_ref, ane_ref, aes_ref,
              w_sn_ref, w_ne_ref, b_ne_ref, w_es_ref, b_es_ref,
              res_n_ref, res_e_ref,
              nout_ref, eout_ref):
    g = g_ref[...]
    e3 = e_ref[...]
    h2c = h2_ref[...]
    swc = sw_ref[...]
    blk = e3.shape[0]
    e = e3.reshape(blk * _NNEI, _EDIM)

    hg_n = _hg(g, h2c, swc, blk, _NDIM)
    sym_pre = _grrg_matmul(hg_n, _NDIM, w_sn_ref, sympre_ref[...])
    node_sym = _silu(sym_pre)

    t_ne = (_mm(g, w_ne_ref[_NDIM:2 * _NDIM])
            + _mm(e, w_ne_ref[2 * _NDIM:]) + b_ne_ref[...])
    arg = ane_ref[...][:, None, :] + t_ne.reshape(blk, _NNEI, _NDIM)
    msg = _silu(arg) * swc.reshape(blk, _NNEI, 1)
    node_edge = jnp.sum(msg, axis=1) * (1.0 / _NNEI)

    t_es = (_mm(g, w_es_ref[_NDIM:2 * _NDIM])
            + _mm(e, w_es_ref[2 * _NDIM:]) + b_es_ref[...])
    earg = aes_ref[...][:, None, :] + t_es.reshape(blk, _NNEI, _EDIM)

    res_n = res_n_ref[...]
    nout_ref[...] = (nacc_ref[...] + res_n[1] * node_sym
                     + res_n[2] * node_edge)
    eout_ref[...] = e3 + res_e_ref[0] * _silu(earg)


def _rows(*tail):
    return pl.BlockSpec((_BLK,) + tail, lambda i: (i,) + (0,) * len(tail))


def _erows(*tail):
    return pl.BlockSpec((_BLK * _NNEI,) + tail,
                        lambda i: (i,) + (0,) * len(tail))


def _whole(x):
    return pl.BlockSpec(x.shape, lambda i: (0,) * x.ndim)


def kernel(node_ebd_ext, edge_ebd, h2, angle_ebd, nlist, nlist_mask, sw,
           a_nlist, a_nlist_mask, a_sw,
           W_self, b_self, W_sym, b_sym, W_ne, b_ne, W_es, b_es, res_n, res_e):
    node = node_ebd_ext[0]
    e = edge_ebd[0]
    h2f = h2[0].reshape(_NLOC * _NNEI, 3)
    swf = sw[0].reshape(_NLOC * _NNEI, 1)
    idx3 = nlist[0].astype(jnp.int32).reshape(_NW, _NG, _CHUNK)

    g4 = _sc_gather(node, idx3)
    g = g4.reshape(_NLOC * _NNEI, _NDIM)

    n_e_sym = _EDIM * _AXIS
    w_se = jnp.stack([W_sym[a:n_e_sym:_AXIS] for a in range(_AXIS)])
    w_sn = jnp.stack([W_sym[n_e_sym + a::_AXIS] for a in range(_AXIS)])

    grid = (_NLOC // _BLK,)
    nacc, sym_pre_e, a_ne, a_es = pl.pallas_call(
        _tc1_body,
        grid=grid,
        in_specs=[
            _rows(_NDIM), _rows(_NNEI, _EDIM), _erows(3), _erows(1),
            _whole(W_self), _whole(b_self.reshape(1, -1)),
            _whole(w_se), _whole(b_sym.reshape(1, -1)),
            _whole(W_ne), _whole(W_es), _whole(res_n),
        ],
        out_specs=[_rows(_NDIM), _rows(_NDIM), _rows(_NDIM), _rows(_EDIM)],
        out_shape=[
            jax.ShapeDtypeStruct((_NLOC, _NDIM), jnp.float32),
            jax.ShapeDtypeStruct((_NLOC, _NDIM), jnp.float32),
            jax.ShapeDtypeStruct((_NLOC, _NDIM), jnp.float32),
            jax.ShapeDtypeStruct((_NLOC, _EDIM), jnp.float32),
        ],
    )(node, e, h2f, swf, W_self, b_self.reshape(1, -1), w_se,
      b_sym.reshape(1, -1), W_ne, W_es, res_n)

    n_out, e_out = pl.pallas_call(
        _tc2_body,
        grid=grid,
        in_specs=[
            _erows(_NDIM), _rows(_NNEI, _EDIM), _erows(3), _erows(1),
            _rows(_NDIM), _rows(_NDIM), _rows(_NDIM), _rows(_EDIM),
            _whole(w_sn), _whole(W_ne), _whole(b_ne.reshape(1, -1)),
            _whole(W_es), _whole(b_es.reshape(1, -1)),
            _whole(res_n), _whole(res_e),
        ],
        out_specs=[_rows(_NDIM), _rows(_NNEI, _EDIM)],
        out_shape=[
            jax.ShapeDtypeStruct((_NLOC, _NDIM), jnp.float32),
            jax.ShapeDtypeStruct((_NLOC, _NNEI, _EDIM), jnp.float32),
        ],
    )(g, e, h2f, swf, nacc, sym_pre_e, a_ne, a_es, w_sn, W_ne,
      b_ne.reshape(1, -1), W_es, b_es.reshape(1, -1), res_n, res_e)

    return n_out[None], e_out[None], angle_ebd


# speedup vs baseline: 1.1916x; 1.1916x over previous
import functools

import jax
import jax.numpy as jnp
from jax import lax
from jax.experimental import pallas as pl
from jax.experimental.pallas import tpu as pltpu
from jax.experimental.pallas import tpu_sc as plsc

_NLOC = 10000
_NNEI = 32
_NDIM = 128
_EDIM = 16
_AXIS = 4

_NW = 32
_ROWS_PER_W = (_NLOC * _NNEI) // _NW
_CHUNK = 80
_NG = _ROWS_PER_W // _CHUNK


def _sc_gather(table, idx3):
    mesh = plsc.VectorSubcoreMesh(core_axis_name="c", subcore_axis_name="s")

    def body(table_hbm, idx_hbm, out_hbm, idx_v, rows_v, gsem, wsem):
        wid = lax.axis_index("s") * 2 + lax.axis_index("c")
        base = wid * _NG
        pltpu.sync_copy(idx_hbm.at[wid], idx_v)
        pltpu.async_copy(table_hbm.at[idx_v.at[0]], rows_v.at[0], gsem)

        def step(g, carry):
            p = lax.rem(g, 2)
            q = lax.rem(g + 1, 2)

            @pl.when(g >= 1)
            def _():
                pltpu.make_async_copy(rows_v.at[q], out_hbm.at[base + g - 1],
                                      wsem).wait()

            @pl.when(g + 1 < _NG)
            def _():
                pltpu.async_copy(table_hbm.at[idx_v.at[g + 1]], rows_v.at[q],
                                 gsem)

            pltpu.make_async_copy(table_hbm.at[idx_v.at[g]], rows_v.at[p],
                                  gsem).wait()
            pltpu.async_copy(rows_v.at[p], out_hbm.at[base + g], wsem)
            return carry

        lax.fori_loop(0, _NG, step, 0, unroll=False)
        last = _NG - 1
        pltpu.make_async_copy(rows_v.at[last % 2], out_hbm.at[base + last],
                              wsem).wait()

    f = pl.kernel(
        body,
        out_type=jax.ShapeDtypeStruct((_NW * _NG, _CHUNK, _NDIM),
                                      jnp.float32),
        mesh=mesh,
        scratch_types=[
            pltpu.VMEM((_NG, _CHUNK), jnp.int32),
            pltpu.VMEM((2, _CHUNK, _NDIM), jnp.float32),
            pltpu.SemaphoreType.DMA,
            pltpu.SemaphoreType.DMA,
        ],
    )
    return f(table, idx3)


def _silu(x):
    return x / (1.0 + jnp.exp(-x))


_BLK = 200


def _mm(a, b):
    return jax.lax.dot_general(a, b, (((a.ndim - 1,), (0,)), ((), ())),
                               preferred_element_type=jnp.float32)


def _hg(ebd, h2c, swc, blk, d):
    out = []
    for t in range(3):
        w_t = h2c[:, t:t + 1] * swc
        out.append(jnp.sum((ebd * w_t).reshape(blk, _NNEI, d), axis=1)
                   * (1.0 / _NNEI))
    return out


def _grrg_matmul(hg, d, w_ref, sym_pre):
    iota = lax.broadcasted_iota(jnp.int32, (d, d), 0)
    for a in range(_AXIS):
        ea = (iota == a).astype(jnp.float32)
        g_a = sum(hg[t] * _mm(hg[t], ea) for t in range(3)) * (1.0 / 3.0)
        sym_pre = sym_pre + _mm(g_a, w_ref[a])
    return sym_pre


def _tc_body(node_ref, g_ref, e_ref, h2_ref, sw_ref,
             w_self_ref, b_self_ref, w_sn_ref, w_se_ref, b_sym_ref,
             w_ne_ref, b_ne_ref, w_es_ref, b_es_ref,
             res_n_ref, res_e_ref,
             nout_ref, eout_ref):
    node = node_ref[...]
    e3 = e_ref[...]
    h2c = h2_ref[...]
    swc = sw_ref[...]
    blk = node.shape[0]
    kb = blk * _NNEI
    g = g_ref[...].reshape(kb, _NDIM)
    e = e3.reshape(kb, _EDIM)

    node_self = _silu(_mm(node, w_self_ref[...]) + b_self_ref[...])

    hg_e = _hg(e, h2c, swc, blk, _EDIM)
    hg_n = _hg(g, h2c, swc, blk, _NDIM)
    sym_pre = _grrg_matmul(hg_e, _EDIM, w_se_ref, b_sym_ref[...])
    sym_pre = _grrg_matmul(hg_n, _NDIM, w_sn_ref, sym_pre)
    node_sym = _silu(sym_pre)

    a_ne = _mm(node, w_ne_ref[0:_NDIM])
    t_ne = (_mm(g, w_ne_ref[_NDIM:2 * _NDIM])
            + _mm(e, w_ne_ref[2 * _NDIM:]) + b_ne_ref[...])
    arg = a_ne[:, None, :] + t_ne.reshape(blk, _NNEI, _NDIM)
    msg = _silu(arg) * swc.reshape(blk, _NNEI, 1)
    node_edge = jnp.sum(msg, axis=1) * (1.0 / _NNEI)

    a_es = _mm(node, w_es_ref[0:_NDIM])
    t_es = (_mm(g, w_es_ref[_NDIM:2 * _NDIM])
            + _mm(e, w_es_ref[2 * _NDIM:]) + b_es_ref[...])
    earg = a_es[:, None, :] + t_es.reshape(blk, _NNEI, _EDIM)

    res_n = res_n_ref[...]
    nout_ref[...] = (node + res_n[0] * node_self + res_n[1] * node_sym
                     + res_n[2] * node_edge)
    eout_ref[...] = e3 + res_e_ref[0] * _silu(earg)


def _rows(*tail):
    return pl.BlockSpec((_BLK,) + tail, lambda i: (i,) + (0,) * len(tail))


def _erows(*tail):
    return pl.BlockSpec((_BLK * _NNEI,) + tail,
                        lambda i: (i,) + (0,) * len(tail))


def _whole(x):
    return pl.BlockSpec(x.shape, lambda i: (0,) * x.ndim)


def kernel(node_ebd_ext, edge_ebd, h2, angle_ebd, nlist, nlist_mask, sw,
           a_nlist, a_nlist_mask, a_sw,
           W_self, b_self, W_sym, b_sym, W_ne, b_ne, W_es, b_es, res_n, res_e):
    node = node_ebd_ext[0]
    e = edge_ebd[0]
    h2f = h2[0].reshape(_NLOC * _NNEI, 3)
    swf = sw[0].reshape(_NLOC * _NNEI, 1)
    idx3 = nlist[0].astype(jnp.int32).reshape(_NW, _NG, _CHUNK)

    g4 = _sc_gather(node, idx3)

    n_e_sym = _EDIM * _AXIS
    w_se = jnp.stack([W_sym[a:n_e_sym:_AXIS] for a in range(_AXIS)])
    w_sn = jnp.stack([W_sym[n_e_sym + a::_AXIS] for a in range(_AXIS)])

    gchunks = (_BLK * _NNEI) // _CHUNK
    grid = (_NLOC // _BLK,)
    n_out, e_out = pl.pallas_call(
        _tc_body,
        grid=grid,
        in_specs=[
            _rows(_NDIM),
            pl.BlockSpec((gchunks, _CHUNK, _NDIM), lambda i: (i, 0, 0)),
            _rows(_NNEI, _EDIM), _erows(3), _erows(1),
            _whole(W_self), _whole(b_self.reshape(1, -1)),
            _whole(w_sn), _whole(w_se), _whole(b_sym.reshape(1, -1)),
            _whole(W_ne), _whole(b_ne.reshape(1, -1)),
            _whole(W_es), _whole(b_es.reshape(1, -1)),
            _whole(res_n), _whole(res_e),
        ],
        out_specs=[_rows(_NDIM), _rows(_NNEI, _EDIM)],
        out_shape=[
            jax.ShapeDtypeStruct((_NLOC, _NDIM), jnp.float32),
            jax.ShapeDtypeStruct((_NLOC, _NNEI, _EDIM), jnp.float32),
        ],
    )(node, g4, e, h2f, swf, W_self, b_self.reshape(1, -1), w_sn, w_se,
      b_sym.reshape(1, -1), W_ne, b_ne.reshape(1, -1), W_es,
      b_es.reshape(1, -1), res_n, res_e)

    return n_out[None], e_out[None], angle_ebd


# speedup vs baseline: 1.1995x; 1.0067x over previous
import functools

import jax
import jax.numpy as jnp
from jax import lax
from jax.experimental import pallas as pl
from jax.experimental.pallas import tpu as pltpu
from jax.experimental.pallas import tpu_sc as plsc

_NLOC = 10000
_NNEI = 32
_NDIM = 128
_EDIM = 16
_AXIS = 4

_NW = 32
_ROWS_PER_W = (_NLOC * _NNEI) // _NW
_CHUNK = 80
_NG = _ROWS_PER_W // _CHUNK


def _sc_gather(table, idx3):
    mesh = plsc.VectorSubcoreMesh(core_axis_name="c", subcore_axis_name="s")

    def body(table_hbm, idx_hbm, out_hbm, idx_v, rows_v, gsem, wsem):
        wid = lax.axis_index("s") * 2 + lax.axis_index("c")
        base = wid * _NG
        pltpu.sync_copy(idx_hbm.at[wid], idx_v)
        pltpu.async_copy(table_hbm.at[idx_v.at[0]], rows_v.at[0], gsem)

        def step(g, carry):
            p = lax.rem(g, 2)
            q = lax.rem(g + 1, 2)

            @pl.when(g >= 1)
            def _():
                pltpu.make_async_copy(rows_v.at[q], out_hbm.at[base + g - 1],
                                      wsem).wait()

            @pl.when(g + 1 < _NG)
            def _():
                pltpu.async_copy(table_hbm.at[idx_v.at[g + 1]], rows_v.at[q],
                                 gsem)

            pltpu.make_async_copy(table_hbm.at[idx_v.at[g]], rows_v.at[p],
                                  gsem).wait()
            pltpu.async_copy(rows_v.at[p], out_hbm.at[base + g], wsem)
            return carry

        lax.fori_loop(0, _NG, step, 0, unroll=False)
        last = _NG - 1
        pltpu.make_async_copy(rows_v.at[last % 2], out_hbm.at[base + last],
                              wsem).wait()

    f = pl.kernel(
        body,
        out_type=jax.ShapeDtypeStruct((_NW * _NG, _CHUNK, _NDIM), jnp.float32),
        mesh=mesh,
        scratch_types=[
            pltpu.VMEM((_NG, _CHUNK), jnp.int32),
            pltpu.VMEM((2, _CHUNK, _NDIM), jnp.float32),
            pltpu.SemaphoreType.DMA,
            pltpu.SemaphoreType.DMA,
        ],
    )
    return f(table, idx3)


def _silu(x):
    return x / (1.0 + jnp.exp(-x))


_BLK = 200


def _mm(a, b):
    return jax.lax.dot_general(a, b, (((a.ndim - 1,), (0,)), ((), ())),
                               preferred_element_type=jnp.float32)


def _hg(ebd, h2c, swc, blk, d):
    out = []
    for t in range(3):
        w_t = h2c[:, t:t + 1] * swc
        out.append(jnp.sum((ebd * w_t).reshape(blk, _NNEI, d), axis=1)
                   * (1.0 / _NNEI))
    return out


def _grrg_matmul(hg, d, w_ref, sym_pre):
    iota = lax.broadcasted_iota(jnp.int32, (d, d), 0)
    for a in range(_AXIS):
        ea = (iota == a).astype(jnp.float32)
        g_a = sum(hg[t] * _mm(hg[t], ea) for t in range(3)) * (1.0 / 3.0)
        sym_pre = sym_pre + _mm(g_a, w_ref[a])
    return sym_pre


def _tc_body(node_ref, g_ref, e_ref, ep_ref, h2k_ref, sw32_ref, h2_ref, sw_ref,
             w_self_ref, b_self_ref, w_sn_ref, w_se_ref, b_sym_ref,
             w_ne_ref, b_ne_ref, w_es_ref, b_es_ref,
             res_n_ref, res_e_ref,
             nout_ref, eout_ref):
    node = node_ref[...]
    e3 = e_ref[...]
    h2c = h2_ref[...]
    swc = sw_ref[...]
    blk = node.shape[0]
    kb = blk * _NNEI
    g = g_ref[...].reshape(kb, _NDIM)
    e = e3.reshape(kb, _EDIM)

    node_self = _silu(_mm(node, w_self_ref[...]) + b_self_ref[...])

    ep = ep_ref[...]
    sw32 = sw32_ref[...]
    iota_r0 = lax.broadcasted_iota(jnp.int32, (_NNEI, _NNEI * _EDIM), 0)
    iota_r1 = lax.broadcasted_iota(jnp.int32, (_NNEI, _NNEI * _EDIM), 1)
    r_exp = (iota_r1 // _EDIM == iota_r0).astype(jnp.float32)
    iota_s0 = lax.broadcasted_iota(jnp.int32, (_NNEI * _EDIM, _EDIM), 0)
    iota_s1 = lax.broadcasted_iota(jnp.int32, (_NNEI * _EDIM, _EDIM), 1)
    r_sum = (iota_s0 % _EDIM == iota_s1).astype(jnp.float32)
    hg_e = []
    for t in range(3):
        w32 = h2k_ref[t] * sw32
        prod = ep * _mm(w32, r_exp)
        hg_e.append(_mm(prod, r_sum) * (1.0 / _NNEI))
    hg_n = _hg(g, h2c, swc, blk, _NDIM)
    sym_pre = _grrg_matmul(hg_e, _EDIM, w_se_ref, b_sym_ref[...])
    sym_pre = _grrg_matmul(hg_n, _NDIM, w_sn_ref, sym_pre)
    node_sym = _silu(sym_pre)

    a_ne = _mm(node, w_ne_ref[0:_NDIM])
    t_ne = (_mm(g, w_ne_ref[_NDIM:2 * _NDIM])
            + _mm(e, w_ne_ref[2 * _NDIM:]) + b_ne_ref[...])
    arg = a_ne[:, None, :] + t_ne.reshape(blk, _NNEI, _NDIM)
    msg = _silu(arg) * swc.reshape(blk, _NNEI, 1)
    node_edge = jnp.sum(msg, axis=1) * (1.0 / _NNEI)

    a_es = _mm(node, w_es_ref[0:_NDIM])
    t_es = (_mm(g, w_es_ref[_NDIM:2 * _NDIM])
            + _mm(e, w_es_ref[2 * _NDIM:]) + b_es_ref[...])
    earg = a_es[:, None, :] + t_es.reshape(blk, _NNEI, _EDIM)

    res_n = res_n_ref[...]
    nout_ref[...] = (node + res_n[0] * node_self + res_n[1] * node_sym
                     + res_n[2] * node_edge)
    eout_ref[...] = e3 + res_e_ref[0] * _silu(earg)


def _rows(*tail):
    return pl.BlockSpec((_BLK,) + tail, lambda i: (i,) + (0,) * len(tail))


def _erows(*tail):
    return pl.BlockSpec((_BLK * _NNEI,) + tail,
                        lambda i: (i,) + (0,) * len(tail))


def _whole(x):
    return pl.BlockSpec(x.shape, lambda i: (0,) * x.ndim)


def kernel(node_ebd_ext, edge_ebd, h2, angle_ebd, nlist, nlist_mask, sw,
           a_nlist, a_nlist_mask, a_sw,
           W_self, b_self, W_sym, b_sym, W_ne, b_ne, W_es, b_es, res_n, res_e):
    node = node_ebd_ext[0]
    e = edge_ebd[0]
    h2f = h2[0].reshape(_NLOC * _NNEI, 3)
    swf = sw[0].reshape(_NLOC * _NNEI, 1)
    ep = edge_ebd[0].reshape(_NLOC, _NNEI * _EDIM)
    h2k = jnp.transpose(h2[0], (2, 0, 1))
    sw32 = sw[0]
    idx3 = nlist[0].astype(jnp.int32).reshape(_NW, _NG, _CHUNK)

    g4 = _sc_gather(node, idx3)

    n_e_sym = _EDIM * _AXIS
    w_se = jnp.stack([W_sym[a:n_e_sym:_AXIS] for a in range(_AXIS)])
    w_sn = jnp.stack([W_sym[n_e_sym + a::_AXIS] for a in range(_AXIS)])

    gchunks = (_BLK * _NNEI) // _CHUNK
    grid = (_NLOC // _BLK,)
    n_out, e_out = pl.pallas_call(
        _tc_body,
        grid=grid,
        in_specs=[
            _rows(_NDIM),
            pl.BlockSpec((gchunks, _CHUNK, _NDIM), lambda i: (i, 0, 0)),
            _rows(_NNEI, _EDIM), _rows(_NNEI * _EDIM),
            pl.BlockSpec((3, _BLK, _NNEI), lambda i: (0, i, 0)),
            _rows(_NNEI), _erows(3), _erows(1),
            _whole(W_self), _whole(b_self.reshape(1, -1)),
            _whole(w_sn), _whole(w_se), _whole(b_sym.reshape(1, -1)),
            _whole(W_ne), _whole(b_ne.reshape(1, -1)),
            _whole(W_es), _whole(b_es.reshape(1, -1)),
            _whole(res_n), _whole(res_e),
        ],
        out_specs=[_rows(_NDIM), _rows(_NNEI, _EDIM)],
        out_shape=[
            jax.ShapeDtypeStruct((_NLOC, _NDIM), jnp.float32),
            jax.ShapeDtypeStruct((_NLOC, _NNEI, _EDIM), jnp.float32),
        ],
    )(node, g4, e, ep, h2k, sw32, h2f, swf, W_self, b_self.reshape(1, -1), w_sn, w_se,
      b_sym.reshape(1, -1), W_ne, b_ne.reshape(1, -1), W_es,
      b_es.reshape(1, -1), res_n, res_e)

    return n_out[None], e_out[None], angle_ebd
